# Initial kernel scaffold; baseline (speedup 1.0000x reference)
import jax
import jax.numpy as jnp
from jax.experimental import pallas as pl


def kernel(pos, attr, edge_index, params):
    raise NotImplementedError("write your pallas kernel here")


# speedup vs baseline: 11.4713x; 11.4713x over previous
import functools
import math

import jax
import jax.numpy as jnp
from jax import lax
from jax.experimental import pallas as pl
from jax.experimental.pallas import tpu as pltpu
from jax.experimental.pallas import tpu_sc as plsc

_CLIP = 60.0


def _conv_pre_body(x_ref, pos_ref, srcw_ref, posw_ref, linw_ref,
                   pb_ref, g_ref, gv_ref, pbo_ref):
    x = x_ref[...]
    p3 = pos_ref[...]
    P = jnp.dot(p3, posw_ref[...], preferred_element_type=jnp.float32)
    B = jnp.dot(x, srcw_ref[...], preferred_element_type=jnp.float32) + P
    V = jnp.dot(x, linw_ref[...], preferred_element_type=jnp.float32)
    g = jnp.exp(jnp.clip(-B, -_CLIP, _CLIP))
    g_ref[...] = g
    gv_ref[...] = g * (V - P)
    pbo_ref[...] = P + pb_ref[...]


def _conv_pre(x, pos, srcw, posw, linw, pb):
    N, fin = x.shape
    W = srcw.shape[1]
    blk = 2000
    nb = N // blk
    grid = (nb,)
    row_spec = lambda c: pl.BlockSpec((blk, c), lambda i: (i, 0))
    full = lambda a: pl.BlockSpec(a.shape, lambda i: (0, 0))
    return pl.pallas_call(
        _conv_pre_body,
        grid=grid,
        in_specs=[row_spec(fin), row_spec(3), full(srcw), full(posw),
                  full(linw), full(pb)],
        out_specs=[row_spec(W), row_spec(W), row_spec(W)],
        out_shape=[jax.ShapeDtypeStruct((N, W), jnp.float32)] * 3,
    )(x, pos, srcw, posw, linw, pb)


def _conv_post_body(ssum_ref, accv_ref, pbo_ref, p1w_ref, p1b_ref,
                    p2w_ref, p2b_ref, y_ref):
    ssum = ssum_ref[...]
    out = (accv_ref[...] + pbo_ref[...] * ssum) / (ssum + 1e-16)
    h = jnp.maximum(
        jnp.dot(out, p1w_ref[...], preferred_element_type=jnp.float32)
        + p1b_ref[...], 0.0)
    y_ref[...] = (jnp.dot(h, p2w_ref[...], preferred_element_type=jnp.float32)
                  + p2b_ref[...])


def _conv_post(ssum, accv, pbo, p1w, p1b, p2w, p2b):
    N, W = ssum.shape
    fout = p2w.shape[1]
    blk = 2000
    nb = N // blk
    row_spec = lambda c: pl.BlockSpec((blk, c), lambda i: (i, 0))
    full = lambda a: pl.BlockSpec(a.shape, lambda i: (0, 0))
    return pl.pallas_call(
        _conv_post_body,
        grid=(nb,),
        in_specs=[row_spec(W), row_spec(W), row_spec(W), full(p1w), full(p1b),
                  full(p2w), full(p2b)],
        out_specs=row_spec(fout),
        out_shape=jax.ShapeDtypeStruct((N, fout), jnp.float32),
    )(ssum, accv, pbo, p1w, p1b, p2w, p2b)


def _sc_edge_segsum(G, GV, src2, dst2, n):
    N, W = G.shape
    NCHUNK, C = src2.shape
    NS = 16
    rpt = n // NS
    ZR = 125
    mesh = plsc.VectorSubcoreMesh(core_axis_name="c", subcore_axis_name="s")

    @functools.partial(
        pl.kernel, mesh=mesh,
        out_type=jax.ShapeDtypeStruct((2, NS, rpt, W), jnp.float32),
        compiler_params=pltpu.CompilerParams(use_tc_tiling_on_sc=False),
        scratch_types=[
            pltpu.VMEM((C,), jnp.int32),
            pltpu.VMEM((C,), jnp.int32),
            pltpu.VMEM((C, W), jnp.float32),
            pltpu.VMEM((ZR, W), jnp.float32),
            pltpu.VMEM_SHARED((n, W), jnp.float32),
            pltpu.SemaphoreType.DMA,
        ])
    def k(g_hbm, gv_hbm, src_hbm, dst_hbm, out_hbm, sidx, didx, rows, zbuf,
          acc, sem):
        cid = lax.axis_index("c")
        sid = lax.axis_index("s")

        def zrow(i, _):
            r = i // (W // 16)
            c = (i % (W // 16)) * 16
            zbuf[r, pl.ds(c, 16)] = jnp.zeros((16,), jnp.float32)
            return 0
        lax.fori_loop(0, ZR * (W // 16), zrow, 0)

        def zacc(j, _):
            pltpu.sync_copy(zbuf, acc.at[pl.ds(sid * rpt + j * ZR, ZR)])
            return 0
        lax.fori_loop(0, rpt // ZR, zacc, 0)
        plsc.subcore_barrier()

        nch = NCHUNK // NS + jnp.where(sid < (NCHUNK % NS), 1, 0)

        def body(j, _):
            ci = sid + j * NS
            pltpu.sync_copy(src_hbm.at[ci], sidx)
            pltpu.sync_copy(dst_hbm.at[ci], didx)

            @pl.when(cid == 0)
            def _():
                pltpu.async_copy(g_hbm.at[sidx], rows, sem).wait()

            @pl.when(cid == 1)
            def _():
                pltpu.async_copy(gv_hbm.at[sidx], rows, sem).wait()

            pltpu.sync_copy(rows, acc.at[didx], add=True)
            return 0
        lax.fori_loop(0, nch, body, 0)
        plsc.subcore_barrier()

        pltpu.sync_copy(acc.at[pl.ds(sid * rpt, rpt)],
                        out_hbm.at[cid, sid])

    return k(G, GV, src2, dst2).reshape(2, n, W)


def _attn_pre_body(x_ref, pos_ref, ln1g_ref, ln1b_ref, pe1w_ref, pe1b_ref,
                   pe2w_ref, pe2b_ref, combwh_ref, combwp_ref, combb_ref,
                   qw_ref, qb_ref, kw_ref, kb_ref, vw_ref, vb_ref,
                   q_ref, k_ref, v_ref):
    x = x_ref[...]
    mu = jnp.mean(x, axis=-1, keepdims=True)
    var = jnp.mean((x - mu) ** 2, axis=-1, keepdims=True)
    h = (x - mu) * jax.lax.rsqrt(var + 1e-5) * ln1g_ref[...] + ln1b_ref[...]
    p3 = pos_ref[...]
    pe = jnp.maximum(
        jnp.dot(p3, pe1w_ref[...], preferred_element_type=jnp.float32)
        + pe1b_ref[...], 0.0)
    pe = jnp.dot(pe, pe2w_ref[...], preferred_element_type=jnp.float32) \
        + pe2b_ref[...]
    comb = (jnp.dot(h, combwh_ref[...], preferred_element_type=jnp.float32)
            + jnp.dot(pe, combwp_ref[...], preferred_element_type=jnp.float32)
            + combb_ref[...])
    q_ref[...] = jnp.dot(comb, qw_ref[...],
                         preferred_element_type=jnp.float32) + qb_ref[...]
    k_ref[...] = jnp.dot(comb, kw_ref[...],
                         preferred_element_type=jnp.float32) + kb_ref[...]
    v_ref[...] = jnp.dot(comb, vw_ref[...],
                         preferred_element_type=jnp.float32) + vb_ref[...]


def _attn_pre(x, pos, p):
    N, dim = x.shape
    pe_dim = p['pe1']['w'].shape[1]
    blk = 2000
    nb = N // blk
    row_spec = lambda c: pl.BlockSpec((blk, c), lambda i: (i, 0))
    full = lambda a: pl.BlockSpec(a.shape, lambda i: (0, 0))
    combwh = p['comb']['w'][:dim]
    combwp = p['comb']['w'][dim:]
    args = (x, pos,
            p['ln1_g'].reshape(1, dim), p['ln1_b'].reshape(1, dim),
            p['pe1']['w'], p['pe1']['b'].reshape(1, pe_dim),
            p['pe2']['w'], p['pe2']['b'].reshape(1, pe_dim),
            combwh, combwp, p['comb']['b'].reshape(1, dim),
            p['q']['w'], p['q']['b'].reshape(1, dim),
            p['k']['w'], p['k']['b'].reshape(1, dim),
            p['v']['w'], p['v']['b'].reshape(1, dim))
    return pl.pallas_call(
        _attn_pre_body,
        grid=(nb,),
        in_specs=[row_spec(dim), row_spec(3)] + [full(a) for a in args[2:]],
        out_specs=[row_spec(dim)] * 3,
        out_shape=[jax.ShapeDtypeStruct((N, dim), jnp.float32)] * 3,
    )(*args)


def _flash_body(q_ref, kt_ref, vt_ref, o_ref, *, scale):
    q = q_ref[0]
    bq, hd = q.shape
    nk = kt_ref.shape[1]

    def step(i, carry):
        m, l, acc = carry
        kt = kt_ref[0, i]
        s = jnp.dot(q, kt, preferred_element_type=jnp.float32) * scale
        m_new = jnp.maximum(m, jnp.max(s, axis=-1, keepdims=True))
        alpha = jnp.exp(m - m_new)
        p = jnp.exp(s - m_new)
        vt = vt_ref[0, i]
        pv = jax.lax.dot_general(p, vt, (((1,), (1,)), ((), ())),
                                 preferred_element_type=jnp.float32)
        l = l * alpha + jnp.sum(p, axis=-1, keepdims=True)
        acc = acc * alpha + pv
        return m_new, l, acc

    m0 = jnp.full((bq, 1), -jnp.inf, jnp.float32)
    l0 = jnp.zeros((bq, 1), jnp.float32)
    a0 = jnp.zeros((bq, hd), jnp.float32)
    m, l, acc = lax.fori_loop(0, nk, step, (m0, l0, a0))
    o_ref[0] = acc / l


def _flash_attn(q, kt, vt, bq=1000, bk=1000):
    H, S, hd = q.shape
    nk = kt.shape[1]
    scale = 1.0 / math.sqrt(hd)
    return pl.pallas_call(
        functools.partial(_flash_body, scale=scale),
        grid=(H, S // bq),
        in_specs=[
            pl.BlockSpec((1, bq, hd), lambda h, i: (h, i, 0)),
            pl.BlockSpec((1, nk, hd, bk), lambda h, i: (h, 0, 0, 0)),
            pl.BlockSpec((1, nk, hd, bk), lambda h, i: (h, 0, 0, 0)),
        ],
        out_specs=pl.BlockSpec((1, bq, hd), lambda h, i: (h, i, 0)),
        out_shape=jax.ShapeDtypeStruct((H, S, hd), jnp.float32),
    )(q, kt, vt)


def _attn_post_body(att_ref, res_ref, ow_ref, ob_ref, ln2g_ref, ln2b_ref,
                    y_ref):
    out = (jnp.dot(att_ref[...], ow_ref[...],
                   preferred_element_type=jnp.float32)
           + ob_ref[...] + res_ref[...])
    mu = jnp.mean(out, axis=-1, keepdims=True)
    var = jnp.mean((out - mu) ** 2, axis=-1, keepdims=True)
    y_ref[...] = ((out - mu) * jax.lax.rsqrt(var + 1e-5) * ln2g_ref[...]
                  + ln2b_ref[...])


def _attn_post(att, res, p):
    N, dim = att.shape
    blk = 2000
    nb = N // blk
    row_spec = pl.BlockSpec((blk, dim), lambda i: (i, 0))
    full = lambda a: pl.BlockSpec(a.shape, lambda i: (0, 0))
    args = (att, res, p['out']['w'], p['out']['b'].reshape(1, dim),
            p['ln2_g'].reshape(1, dim), p['ln2_b'].reshape(1, dim))
    return pl.pallas_call(
        _attn_post_body,
        grid=(nb,),
        in_specs=[row_spec, row_spec] + [full(a) for a in args[2:]],
        out_specs=row_spec,
        out_shape=jax.ShapeDtypeStruct((N, dim), jnp.float32),
    )(*args)


def _stack_conv_weights(p, fin):
    convs = p['convs']
    srcw = jnp.concatenate([c['src'] for c in convs], axis=1)
    linw = jnp.concatenate([c['lin'] for c in convs], axis=1)
    posw = jnp.concatenate([c['pos']['w'] for c in convs], axis=1)
    posb = jnp.concatenate([c['pos']['b'] for c in convs], axis=0)
    return srcw, posw, linw, posb.reshape(1, -1)


def _mh_pt_layer(x, pos, src2, dst2, n, p):
    fin = x.shape[1]
    srcw, posw, linw, posb = _stack_conv_weights(p, fin)
    G, GV, Pbo = _conv_pre(x, pos, srcw, posw, linw, posb)
    W = G.shape[1]
    if W <= 96:
        acc = _sc_edge_segsum(G, GV, src2, dst2, n)
        ssum, accv = acc[0], acc[1]
    else:
        h = W // 2
        a1 = _sc_edge_segsum(G[:, :h], G[:, h:], src2, dst2, n)
        a2 = _sc_edge_segsum(GV[:, :h], GV[:, h:], src2, dst2, n)
        ssum = jnp.concatenate([a1[0], a1[1]], axis=1)
        accv = jnp.concatenate([a2[0], a2[1]], axis=1)
    return _conv_post(ssum, accv, Pbo,
                      p['proj1']['w'], p['proj1']['b'].reshape(1, -1),
                      p['proj2']['w'], p['proj2']['b'].reshape(1, -1))


def _attn_layer(x, pos, p, num_heads):
    n, dim = x.shape
    hd = dim // num_heads
    q, k, v = _attn_pre(x, pos, p)
    bk = 1000
    nk = n // bk
    qh = q.reshape(n, num_heads, hd).transpose(1, 0, 2)
    kt = (k.reshape(n, num_heads, hd).transpose(1, 2, 0)
          .reshape(num_heads, hd, nk, bk).transpose(0, 2, 1, 3))
    vt = (v.reshape(n, num_heads, hd).transpose(1, 2, 0)
          .reshape(num_heads, hd, nk, bk).transpose(0, 2, 1, 3))
    o = _flash_attn(qh, kt, vt, bk=bk)
    att = o.transpose(1, 0, 2).reshape(n, dim)
    return _attn_post(att, x, p)


def kernel(pos, attr, edge_index, params):
    n = pos.shape[0]
    src = edge_index[0].astype(jnp.int32)
    dst = edge_index[1].astype(jnp.int32)
    C = 128
    src2 = src.reshape(-1, C)
    dst2 = dst.reshape(-1, C)

    x = _mh_pt_layer(attr, pos, src2, dst2, n, params['fe_pt1'])
    x = _attn_layer(x, pos, params['fe_attn'], 4)
    x = _mh_pt_layer(x, pos, src2, dst2, n, params['fe_pt2'])
    xe = _mh_pt_layer(x, pos, src2, dst2, n, params['exp_pt'])

    Cdim, r = 96, 2
    xu = jnp.transpose(xe.reshape(n, r, Cdim), (1, 0, 2)).reshape(r * n, Cdim)
    pos_rn = jnp.tile(pos[None, :, :], (r, 1, 1)).reshape(r * n, 3)
    return _attn_layer(xu, pos_rn, params['exp_attn'], 2)
